# baseline (speedup 1.0000x reference)
import functools

import jax
import jax.numpy as jnp
from jax import lax
from jax.experimental import pallas as pl
from jax.experimental.pallas import tpu as pltpu
from jax.experimental.pallas import tpu_sc as plsc

N = 10000
E = 320000
D = 128
H = 128
O = 16

NC = 2
NS = 16
EPC = E // NC
EPW = EPC // NS
CH = 40
FIRE = 5
NITER = EPW // (CH * FIRE)
NCH = EPW // CH
NPAD = 10240
RPT = NPAD // NS
RLAST = N - (NS - 1) * RPT


def _sc_mesh():
    return plsc.VectorSubcoreMesh(core_axis_name="c", subcore_axis_name="s")


@functools.lru_cache(maxsize=None)
def _deg_kernel():
    @functools.partial(
        pl.kernel,
        out_type=jax.ShapeDtypeStruct((NC, N), jnp.float32),
        mesh=_sc_mesh(),
        scratch_types=[
            [pltpu.VMEM((CH,), jnp.int32)] * FIRE,
            pltpu.VMEM((48,), jnp.float32),
            pltpu.VMEM_SHARED((N,), jnp.float32),
            [pltpu.SemaphoreType.DMA] * FIRE,
        ],
    )
    def deg_k(dst_hbm, zeros_hbm, out_hbm, didx_v, ones_v, acc_sh, dsems):
        cid = lax.axis_index("c")
        sid = lax.axis_index("s")
        wid = cid * NS + sid
        for k in range(3):
            ones_v[pl.ds(k * 16, 16)] = jnp.ones((16,), jnp.float32)
        @pl.when(sid == 0)
        def _zero():
            pltpu.sync_copy(zeros_hbm, acc_sh)

        plsc.subcore_barrier()
        ebase = wid * EPW

        def body(g, carry):
            base = ebase + g * (FIRE * CH)
            ddescs = [
                pltpu.async_copy(dst_hbm.at[pl.ds(base + b * CH, CH)],
                                 didx_v[b], dsems[b])
                for b in range(FIRE)
            ]
            for b in range(FIRE):
                ddescs[b].wait()
                pltpu.sync_copy(ones_v.at[pl.ds(0, CH)], acc_sh.at[didx_v[b]],
                                add=True)
            return carry

        lax.fori_loop(0, NCH // FIRE, body, 0)
        plsc.subcore_barrier()

        @pl.when(sid == 0)
        def _out():
            pltpu.sync_copy(acc_sh, out_hbm.at[cid])

    return deg_k


@functools.lru_cache(maxsize=None)
def _agg_kernel(w, ch):
    niter = EPW // (ch * FIRE)

    @functools.partial(
        pl.kernel,
        out_type=jax.ShapeDtypeStruct((NC, N, w), jnp.float32),
        mesh=_sc_mesh(),
        scratch_types=[
            pltpu.VMEM((FIRE * ch,), jnp.int32),
            [pltpu.VMEM((ch,), jnp.int32)] * FIRE,
            [pltpu.VMEM((ch, w), jnp.float32)] * FIRE,
            pltpu.VMEM_SHARED((NPAD, w), jnp.float32),
            [pltpu.SemaphoreType.DMA] * FIRE,
            [pltpu.SemaphoreType.DMA] * FIRE,
            pltpu.SemaphoreType.DMA,
        ],
    )
    def agg_k(q_hbm, src_hbm, dst_hbm, zeros_hbm, out_hbm,
              sidx_v, didx_v, rows_v, acc_sh, sems, dsems, ssem):
        cid = lax.axis_index("c")
        sid = lax.axis_index("s")
        wid = cid * NS + sid
        r0 = pl.multiple_of(sid * RPT, 8)
        ebase = wid * EPW
        pltpu.sync_copy(zeros_hbm.at[sid], acc_sh.at[pl.ds(r0, RPT)])
        plsc.subcore_barrier()

        def body(g, carry):
            base = ebase + g * (FIRE * ch)
            sdesc = pltpu.async_copy(src_hbm.at[pl.ds(base, FIRE * ch)],
                                     sidx_v, ssem)
            ddescs = [
                pltpu.async_copy(dst_hbm.at[pl.ds(base + b * ch, ch)],
                                 didx_v[b], dsems[b])
                for b in range(FIRE)
            ]
            sdesc.wait()
            descs = [
                pltpu.async_copy(
                    q_hbm.at[sidx_v.at[pl.ds(b * ch, ch)]], rows_v[b], sems[b])
                for b in range(FIRE)
            ]
            for b in range(FIRE):
                ddescs[b].wait()
                descs[b].wait()
                pltpu.sync_copy(rows_v[b], acc_sh.at[didx_v[b]], add=True)
            return carry

        lax.fori_loop(0, niter, body, 0)
        plsc.subcore_barrier()

        @pl.when(sid < NS - 1)
        def _out_full():
            pltpu.sync_copy(acc_sh.at[pl.ds(r0, RPT)],
                            out_hbm.at[cid, pl.ds(r0, RPT)])

        @pl.when(sid == NS - 1)
        def _out_last():
            pltpu.sync_copy(acc_sh.at[pl.ds((NS - 1) * RPT, RLAST)],
                            out_hbm.at[cid, pl.ds((NS - 1) * RPT, RLAST)])

    return agg_k


BR = 400
GR = N // BR

BM = 1024
GM = (N + BM - 1) // BM


def _k1_body(x_b, mlpW, mlpb, W1, degp, q1_b, dis_b):
    h0 = jnp.dot(x_b[...], mlpW[...], preferred_element_type=jnp.float32)
    h0 = h0 + mlpb[...]
    deg = degp[...][:, 0] + degp[...][:, 1] + 1.0
    dis = lax.rsqrt(deg)
    q1_b[...] = jnp.dot(h0, W1[...], preferred_element_type=jnp.float32) * dis[:, None]
    dis_b[...] = dis[:, None]


def _k2_body(s_b, q_b, dis_b, bias, W, q2_b):
    agg = s_b[...][0] + s_b[...][1] + q_b[...]
    dis = dis_b[...]
    h = jnp.maximum(agg * dis + bias[...], 0.0)
    q2_b[...] = jnp.dot(h, W[...], preferred_element_type=jnp.float32) * dis


def _k3_body(s_b, q_b, dis_b, bias, u_b):
    agg = s_b[...][0] + s_b[...][1] + q_b[...]
    dis = dis_b[...]
    h = jnp.maximum(agg * dis + bias[...], 0.0)
    u_b[...] = h * dis


def _k4_body(s_b, u_b, dis_b, Wm, bm, Ws, bs, noise_b, z_b):
    g = (s_b[...][0] + s_b[...][1] + u_b[...]) * dis_b[...]
    mean = jnp.maximum(
        jnp.dot(g, Wm[...], preferred_element_type=jnp.float32) + bm[...], 0.0)
    logstd = jnp.maximum(
        jnp.dot(g, Ws[...], preferred_element_type=jnp.float32) + bs[...], 0.0)
    z_b[...] = noise_b[...] * jnp.exp(logstd) + mean


def _k5_body(z_b, zT_b, out_b):
    i = pl.program_id(0)
    j = pl.program_id(1)
    row0 = i * BM
    col0 = j * BM

    @pl.when(col0 + BM - 1 <= row0)
    def _zeros():
        out_b[...] = jnp.zeros((BM, BM), jnp.float32)

    @pl.when(col0 + BM - 1 > row0)
    def _compute():
        a = jnp.dot(z_b[...], zT_b[...], preferred_element_type=jnp.float32)
        row = row0 + lax.broadcasted_iota(jnp.int32, (BM, BM), 0)
        col = col0 + lax.broadcasted_iota(jnp.int32, (BM, BM), 1)
        out_b[...] = jnp.where(col > row, jax.nn.sigmoid(a), 0.0)


def _row_spec(w):
    return pl.BlockSpec((BR, w), lambda i: (i, 0))


def _full_spec(shape):
    nd = len(shape)
    return pl.BlockSpec(shape, lambda i: (0,) * nd)


def kernel(x, edge_index, mlp_W, mlp_b, W1, b1, W2, b2, Wm, bm, Ws, bs, noise):
    f32 = jnp.float32
    src = edge_index[0].astype(jnp.int32)
    dst = edge_index[1].astype(jnp.int32)
    mlp_b2 = mlp_b.reshape(1, H)
    b1_2 = b1.reshape(1, H)
    b2_2 = b2.reshape(1, H)
    bm_2 = bm.reshape(1, O)
    bs_2 = bs.reshape(1, O)
    zeros1 = jnp.zeros((N,), f32)
    zeros3 = jnp.zeros((NS, RPT, H), f32)

    deg_parts = _deg_kernel()(dst, zeros1)
    degp = deg_parts.T

    agg = _agg_kernel(H, CH)

    q1, dis = pl.pallas_call(
        _k1_body,
        grid=(GR,),
        in_specs=[_row_spec(D), _full_spec((D, H)), _full_spec((1, H)),
                  _full_spec((H, H)), pl.BlockSpec((BR, NC), lambda i: (i, 0))],
        out_specs=[_row_spec(H), pl.BlockSpec((BR, 1), lambda i: (i, 0))],
        out_shape=[jax.ShapeDtypeStruct((N, H), f32),
                   jax.ShapeDtypeStruct((N, 1), f32)],
    )(x, mlp_W, mlp_b2, W1, degp)

    s1 = agg(q1, src, dst, zeros3)

    def stage2(s, q, bias, W):
        return pl.pallas_call(
            _k2_body,
            grid=(GR,),
            in_specs=[pl.BlockSpec((NC, BR, H), lambda i: (0, i, 0)),
                      _row_spec(H), pl.BlockSpec((BR, 1), lambda i: (i, 0)),
                      _full_spec((1, H)), _full_spec((H, H))],
            out_specs=_row_spec(H),
            out_shape=jax.ShapeDtypeStruct((N, H), f32),
        )(s, q, dis, bias, W)

    q2 = stage2(s1, q1, b1_2, W2)
    s2 = agg(q2, src, dst, zeros3)

    u = pl.pallas_call(
        _k3_body,
        grid=(GR,),
        in_specs=[pl.BlockSpec((NC, BR, H), lambda i: (0, i, 0)),
                  _row_spec(H), pl.BlockSpec((BR, 1), lambda i: (i, 0)),
                  _full_spec((1, H))],
        out_specs=_row_spec(H),
        out_shape=jax.ShapeDtypeStruct((N, H), f32),
    )(s2, q2, dis, b2_2)

    s3 = agg(u, src, dst, zeros3)

    z = pl.pallas_call(
        _k4_body,
        grid=(GR,),
        in_specs=[pl.BlockSpec((NC, BR, H), lambda i: (0, i, 0)),
                  _row_spec(H), pl.BlockSpec((BR, 1), lambda i: (i, 0)),
                  _full_spec((H, O)), _full_spec((1, O)),
                  _full_spec((H, O)), _full_spec((1, O)),
                  _row_spec(O)],
        out_specs=_row_spec(O),
        out_shape=jax.ShapeDtypeStruct((N, O), f32),
    )(s3, u, dis, Wm, bm_2, Ws, bs_2, noise)

    zT = z.T

    adj = pl.pallas_call(
        _k5_body,
        grid=(GM, GM),
        in_specs=[pl.BlockSpec((BM, O), lambda i, j: (i, 0)),
                  pl.BlockSpec((O, BM), lambda i, j: (0, j))],
        out_specs=pl.BlockSpec((BM, BM), lambda i, j: (i, j)),
        out_shape=jax.ShapeDtypeStruct((N, N), f32),
    )(z, zT)

    return adj


# speedup vs baseline: 1.0221x; 1.0221x over previous
import functools

import jax
import jax.numpy as jnp
from jax import lax
from jax.experimental import pallas as pl
from jax.experimental.pallas import tpu as pltpu
from jax.experimental.pallas import tpu_sc as plsc

N = 10000
E = 320000
D = 128
H = 128
O = 16

NC = 2
NS = 16
EPC = E // NC
EPW = EPC // NS
CH = 40
FIRE = 5
NITER = EPW // (CH * FIRE)
NCH = EPW // CH
NPAD = 10240
RPT = NPAD // NS
RLAST = N - (NS - 1) * RPT


def _sc_mesh():
    return plsc.VectorSubcoreMesh(core_axis_name="c", subcore_axis_name="s")


@functools.lru_cache(maxsize=None)
def _deg_kernel():
    @functools.partial(
        pl.kernel,
        out_type=jax.ShapeDtypeStruct((NC, N), jnp.float32),
        mesh=_sc_mesh(),
        scratch_types=[
            [pltpu.VMEM((CH,), jnp.int32)] * FIRE,
            pltpu.VMEM((48,), jnp.float32),
            pltpu.VMEM_SHARED((N,), jnp.float32),
            [pltpu.SemaphoreType.DMA] * FIRE,
        ],
    )
    def deg_k(dst_hbm, zeros_hbm, out_hbm, didx_v, ones_v, acc_sh, dsems):
        cid = lax.axis_index("c")
        sid = lax.axis_index("s")
        wid = cid * NS + sid
        for k in range(3):
            ones_v[pl.ds(k * 16, 16)] = jnp.ones((16,), jnp.float32)
        @pl.when(sid == 0)
        def _zero():
            pltpu.sync_copy(zeros_hbm, acc_sh)

        plsc.subcore_barrier()
        ebase = wid * EPW

        def body(g, carry):
            base = ebase + g * (FIRE * CH)
            ddescs = [
                pltpu.async_copy(dst_hbm.at[pl.ds(base + b * CH, CH)],
                                 didx_v[b], dsems[b])
                for b in range(FIRE)
            ]
            for b in range(FIRE):
                ddescs[b].wait()
                pltpu.sync_copy(ones_v.at[pl.ds(0, CH)], acc_sh.at[didx_v[b]],
                                add=True)
            return carry

        lax.fori_loop(0, NCH // FIRE, body, 0)
        plsc.subcore_barrier()

        @pl.when(sid == 0)
        def _out():
            pltpu.sync_copy(acc_sh, out_hbm.at[cid])

    return deg_k


@functools.lru_cache(maxsize=None)
def _agg_kernel(w, ch):
    niter = EPW // (ch * FIRE)

    @functools.partial(
        pl.kernel,
        out_type=jax.ShapeDtypeStruct((NC, N, w), jnp.float32),
        mesh=_sc_mesh(),
        scratch_types=[
            pltpu.VMEM((EPW,), jnp.int32),
            [pltpu.VMEM((ch,), jnp.int32)] * FIRE,
            [pltpu.VMEM((ch, w), jnp.float32)] * FIRE,
            pltpu.VMEM_SHARED((NPAD, w), jnp.float32),
            [pltpu.SemaphoreType.DMA] * FIRE,
            [pltpu.SemaphoreType.DMA] * FIRE,
            pltpu.SemaphoreType.DMA,
        ],
    )
    def agg_k(q_hbm, src_hbm, dst_hbm, zeros_hbm, out_hbm,
              sidx_v, didx_v, rows_v, acc_sh, sems, dsems, ssem):
        cid = lax.axis_index("c")
        sid = lax.axis_index("s")
        wid = cid * NS + sid
        r0 = pl.multiple_of(sid * RPT, 8)
        ebase = wid * EPW
        pltpu.sync_copy(zeros_hbm.at[sid], acc_sh.at[pl.ds(r0, RPT)])
        plsc.subcore_barrier()

        pltpu.sync_copy(src_hbm.at[pl.ds(ebase, EPW)], sidx_v)

        def body(g, carry):
            base = ebase + g * (FIRE * ch)
            ddescs = [
                pltpu.async_copy(dst_hbm.at[pl.ds(base + b * ch, ch)],
                                 didx_v[b], dsems[b])
                for b in range(FIRE)
            ]
            descs = [
                pltpu.async_copy(
                    q_hbm.at[sidx_v.at[pl.ds((g * FIRE + b) * ch, ch)]],
                    rows_v[b], sems[b])
                for b in range(FIRE)
            ]
            for b in range(FIRE):
                ddescs[b].wait()
                descs[b].wait()
                pltpu.sync_copy(rows_v[b], acc_sh.at[didx_v[b]], add=True)
            return carry

        lax.fori_loop(0, niter, body, 0)
        plsc.subcore_barrier()

        @pl.when(sid < NS - 1)
        def _out_full():
            pltpu.sync_copy(acc_sh.at[pl.ds(r0, RPT)],
                            out_hbm.at[cid, pl.ds(r0, RPT)])

        @pl.when(sid == NS - 1)
        def _out_last():
            pltpu.sync_copy(acc_sh.at[pl.ds((NS - 1) * RPT, RLAST)],
                            out_hbm.at[cid, pl.ds((NS - 1) * RPT, RLAST)])

    return agg_k


BR = 400
GR = N // BR

BM = 1024
GM = (N + BM - 1) // BM


def _k1_body(x_b, mlpW, mlpb, W1, degp, q1_b, dis_b):
    h0 = jnp.dot(x_b[...], mlpW[...], preferred_element_type=jnp.float32)
    h0 = h0 + mlpb[...]
    deg = degp[...][:, 0] + degp[...][:, 1] + 1.0
    dis = lax.rsqrt(deg)
    q1_b[...] = jnp.dot(h0, W1[...], preferred_element_type=jnp.float32) * dis[:, None]
    dis_b[...] = dis[:, None]


def _k2_body(s_b, q_b, dis_b, bias, W, q2_b):
    agg = s_b[...][0] + s_b[...][1] + q_b[...]
    dis = dis_b[...]
    h = jnp.maximum(agg * dis + bias[...], 0.0)
    q2_b[...] = jnp.dot(h, W[...], preferred_element_type=jnp.float32) * dis


def _k3_body(s_b, q_b, dis_b, bias, u_b):
    agg = s_b[...][0] + s_b[...][1] + q_b[...]
    dis = dis_b[...]
    h = jnp.maximum(agg * dis + bias[...], 0.0)
    u_b[...] = h * dis


def _k4_body(s_b, u_b, dis_b, Wm, bm, Ws, bs, noise_b, z_b):
    g = (s_b[...][0] + s_b[...][1] + u_b[...]) * dis_b[...]
    mean = jnp.maximum(
        jnp.dot(g, Wm[...], preferred_element_type=jnp.float32) + bm[...], 0.0)
    logstd = jnp.maximum(
        jnp.dot(g, Ws[...], preferred_element_type=jnp.float32) + bs[...], 0.0)
    z_b[...] = noise_b[...] * jnp.exp(logstd) + mean


def _k5_body(z_b, zT_b, out_b):
    i = pl.program_id(0)
    j = pl.program_id(1)
    row0 = i * BM
    col0 = j * BM

    @pl.when(col0 + BM - 1 <= row0)
    def _zeros():
        out_b[...] = jnp.zeros((BM, BM), jnp.float32)

    @pl.when(col0 + BM - 1 > row0)
    def _compute():
        a = jnp.dot(z_b[...], zT_b[...], preferred_element_type=jnp.float32)
        row = row0 + lax.broadcasted_iota(jnp.int32, (BM, BM), 0)
        col = col0 + lax.broadcasted_iota(jnp.int32, (BM, BM), 1)
        out_b[...] = jnp.where(col > row, jax.nn.sigmoid(a), 0.0)


def _row_spec(w):
    return pl.BlockSpec((BR, w), lambda i: (i, 0))


def _full_spec(shape):
    nd = len(shape)
    return pl.BlockSpec(shape, lambda i: (0,) * nd)


def kernel(x, edge_index, mlp_W, mlp_b, W1, b1, W2, b2, Wm, bm, Ws, bs, noise):
    f32 = jnp.float32
    src = edge_index[0].astype(jnp.int32)
    dst = edge_index[1].astype(jnp.int32)
    mlp_b2 = mlp_b.reshape(1, H)
    b1_2 = b1.reshape(1, H)
    b2_2 = b2.reshape(1, H)
    bm_2 = bm.reshape(1, O)
    bs_2 = bs.reshape(1, O)
    zeros1 = jnp.zeros((N,), f32)
    zeros3 = jnp.zeros((NS, RPT, H), f32)

    deg_parts = _deg_kernel()(dst, zeros1)
    degp = deg_parts.T

    agg = _agg_kernel(H, CH)

    q1, dis = pl.pallas_call(
        _k1_body,
        grid=(GR,),
        in_specs=[_row_spec(D), _full_spec((D, H)), _full_spec((1, H)),
                  _full_spec((H, H)), pl.BlockSpec((BR, NC), lambda i: (i, 0))],
        out_specs=[_row_spec(H), pl.BlockSpec((BR, 1), lambda i: (i, 0))],
        out_shape=[jax.ShapeDtypeStruct((N, H), f32),
                   jax.ShapeDtypeStruct((N, 1), f32)],
    )(x, mlp_W, mlp_b2, W1, degp)

    s1 = agg(q1, src, dst, zeros3)

    def stage2(s, q, bias, W):
        return pl.pallas_call(
            _k2_body,
            grid=(GR,),
            in_specs=[pl.BlockSpec((NC, BR, H), lambda i: (0, i, 0)),
                      _row_spec(H), pl.BlockSpec((BR, 1), lambda i: (i, 0)),
                      _full_spec((1, H)), _full_spec((H, H))],
            out_specs=_row_spec(H),
            out_shape=jax.ShapeDtypeStruct((N, H), f32),
        )(s, q, dis, bias, W)

    q2 = stage2(s1, q1, b1_2, W2)
    s2 = agg(q2, src, dst, zeros3)

    u = pl.pallas_call(
        _k3_body,
        grid=(GR,),
        in_specs=[pl.BlockSpec((NC, BR, H), lambda i: (0, i, 0)),
                  _row_spec(H), pl.BlockSpec((BR, 1), lambda i: (i, 0)),
                  _full_spec((1, H))],
        out_specs=_row_spec(H),
        out_shape=jax.ShapeDtypeStruct((N, H), f32),
    )(s2, q2, dis, b2_2)

    s3 = agg(u, src, dst, zeros3)

    z = pl.pallas_call(
        _k4_body,
        grid=(GR,),
        in_specs=[pl.BlockSpec((NC, BR, H), lambda i: (0, i, 0)),
                  _row_spec(H), pl.BlockSpec((BR, 1), lambda i: (i, 0)),
                  _full_spec((H, O)), _full_spec((1, O)),
                  _full_spec((H, O)), _full_spec((1, O)),
                  _row_spec(O)],
        out_specs=_row_spec(O),
        out_shape=jax.ShapeDtypeStruct((N, O), f32),
    )(s3, u, dis, Wm, bm_2, Ws, bs_2, noise)

    zT = z.T

    adj = pl.pallas_call(
        _k5_body,
        grid=(GM, GM),
        in_specs=[pl.BlockSpec((BM, O), lambda i, j: (i, 0)),
                  pl.BlockSpec((O, BM), lambda i, j: (0, j))],
        out_specs=pl.BlockSpec((BM, BM), lambda i, j: (i, j)),
        out_shape=jax.ShapeDtypeStruct((N, N), f32),
    )(z, zT)

    return adj


# speedup vs baseline: 1.0850x; 1.0615x over previous
import functools

import jax
import jax.numpy as jnp
from jax import lax
from jax.experimental import pallas as pl
from jax.experimental.pallas import tpu as pltpu
from jax.experimental.pallas import tpu_sc as plsc

N = 10000
E = 320000
D = 128
H = 128
O = 16

NC = 2
NS = 16
EPC = E // NC
EPW = EPC // NS
CH = 40
FIRE = 5
NITER = EPW // (CH * FIRE)
NCH = EPW // CH
NPAD = 10240
RPT = NPAD // NS
RLAST = N - (NS - 1) * RPT


def _sc_mesh():
    return plsc.VectorSubcoreMesh(core_axis_name="c", subcore_axis_name="s")


@functools.lru_cache(maxsize=None)
def _deg_kernel():
    @functools.partial(
        pl.kernel,
        out_type=jax.ShapeDtypeStruct((NC, N), jnp.float32),
        mesh=_sc_mesh(),
        scratch_types=[
            [pltpu.VMEM((CH,), jnp.int32)] * FIRE,
            pltpu.VMEM((48,), jnp.float32),
            pltpu.VMEM_SHARED((N,), jnp.float32),
            [pltpu.SemaphoreType.DMA] * FIRE,
        ],
    )
    def deg_k(dst_hbm, zeros_hbm, out_hbm, didx_v, ones_v, acc_sh, dsems):
        cid = lax.axis_index("c")
        sid = lax.axis_index("s")
        wid = cid * NS + sid
        for k in range(3):
            ones_v[pl.ds(k * 16, 16)] = jnp.ones((16,), jnp.float32)
        @pl.when(sid == 0)
        def _zero():
            pltpu.sync_copy(zeros_hbm, acc_sh)

        plsc.subcore_barrier()
        ebase = wid * EPW

        def body(g, carry):
            base = ebase + g * (FIRE * CH)
            ddescs = [
                pltpu.async_copy(dst_hbm.at[pl.ds(base + b * CH, CH)],
                                 didx_v[b], dsems[b])
                for b in range(FIRE)
            ]
            for b in range(FIRE):
                ddescs[b].wait()
                pltpu.sync_copy(ones_v.at[pl.ds(0, CH)], acc_sh.at[didx_v[b]],
                                add=True)
            return carry

        lax.fori_loop(0, NCH // FIRE, body, 0)
        plsc.subcore_barrier()

        @pl.when(sid == 0)
        def _out():
            pltpu.sync_copy(acc_sh, out_hbm.at[cid])

    return deg_k


@functools.lru_cache(maxsize=None)
def _agg_kernel(w, ch):
    niter = EPW // (ch * FIRE)

    @functools.partial(
        pl.kernel,
        out_type=jax.ShapeDtypeStruct((NC, N, w), jnp.float32),
        mesh=_sc_mesh(),
        scratch_types=[
            pltpu.VMEM((EPW,), jnp.int32),
            pltpu.VMEM((EPW,), jnp.int32),
            [pltpu.VMEM((ch, w), jnp.float32)] * FIRE,
            pltpu.VMEM_SHARED((NPAD, w), jnp.float32),
            [pltpu.SemaphoreType.DMA] * FIRE,
            [pltpu.SemaphoreType.DMA] * FIRE,
            pltpu.SemaphoreType.DMA,
        ],
    )
    def agg_k(q_hbm, src_hbm, dst_hbm, zeros_hbm, out_hbm,
              sidx_v, didx_v, rows_v, acc_sh, sems, dsems, ssem):
        cid = lax.axis_index("c")
        sid = lax.axis_index("s")
        wid = cid * NS + sid
        r0 = pl.multiple_of(sid * RPT, 8)
        ebase = wid * EPW
        pltpu.sync_copy(zeros_hbm.at[sid], acc_sh.at[pl.ds(r0, RPT)])
        plsc.subcore_barrier()

        pltpu.sync_copy(src_hbm.at[pl.ds(ebase, EPW)], sidx_v)
        pltpu.sync_copy(dst_hbm.at[pl.ds(ebase, EPW)], didx_v)

        def body(g, carry):
            descs = [
                pltpu.async_copy(
                    q_hbm.at[sidx_v.at[pl.ds((g * FIRE + b) * ch, ch)]],
                    rows_v[b], sems[b])
                for b in range(FIRE)
            ]
            for b in range(FIRE):
                descs[b].wait()
                pltpu.sync_copy(
                    rows_v[b],
                    acc_sh.at[didx_v.at[pl.ds((g * FIRE + b) * ch, ch)]],
                    add=True)
            return carry

        lax.fori_loop(0, niter, body, 0)
        plsc.subcore_barrier()

        @pl.when(sid < NS - 1)
        def _out_full():
            pltpu.sync_copy(acc_sh.at[pl.ds(r0, RPT)],
                            out_hbm.at[cid, pl.ds(r0, RPT)])

        @pl.when(sid == NS - 1)
        def _out_last():
            pltpu.sync_copy(acc_sh.at[pl.ds((NS - 1) * RPT, RLAST)],
                            out_hbm.at[cid, pl.ds((NS - 1) * RPT, RLAST)])

    return agg_k


BR = 400
GR = N // BR

BM = 1024
GM = (N + BM - 1) // BM


def _k1_body(x_b, mlpW, mlpb, W1, degp, q1_b, dis_b):
    h0 = jnp.dot(x_b[...], mlpW[...], preferred_element_type=jnp.float32)
    h0 = h0 + mlpb[...]
    deg = degp[...][:, 0] + degp[...][:, 1] + 1.0
    dis = lax.rsqrt(deg)
    q1_b[...] = jnp.dot(h0, W1[...], preferred_element_type=jnp.float32) * dis[:, None]
    dis_b[...] = dis[:, None]


def _k2_body(s_b, q_b, dis_b, bias, W, q2_b):
    agg = s_b[...][0] + s_b[...][1] + q_b[...]
    dis = dis_b[...]
    h = jnp.maximum(agg * dis + bias[...], 0.0)
    q2_b[...] = jnp.dot(h, W[...], preferred_element_type=jnp.float32) * dis


def _k3_body(s_b, q_b, dis_b, bias, u_b):
    agg = s_b[...][0] + s_b[...][1] + q_b[...]
    dis = dis_b[...]
    h = jnp.maximum(agg * dis + bias[...], 0.0)
    u_b[...] = h * dis


def _k4_body(s_b, u_b, dis_b, Wm, bm, Ws, bs, noise_b, z_b):
    g = (s_b[...][0] + s_b[...][1] + u_b[...]) * dis_b[...]
    mean = jnp.maximum(
        jnp.dot(g, Wm[...], preferred_element_type=jnp.float32) + bm[...], 0.0)
    logstd = jnp.maximum(
        jnp.dot(g, Ws[...], preferred_element_type=jnp.float32) + bs[...], 0.0)
    z_b[...] = noise_b[...] * jnp.exp(logstd) + mean


def _k5_body(z_b, zT_b, out_b):
    i = pl.program_id(0)
    j = pl.program_id(1)
    row0 = i * BM
    col0 = j * BM

    @pl.when(col0 + BM - 1 <= row0)
    def _zeros():
        out_b[...] = jnp.zeros((BM, BM), jnp.float32)

    @pl.when(col0 + BM - 1 > row0)
    def _compute():
        a = jnp.dot(z_b[...], zT_b[...], preferred_element_type=jnp.float32)
        row = row0 + lax.broadcasted_iota(jnp.int32, (BM, BM), 0)
        col = col0 + lax.broadcasted_iota(jnp.int32, (BM, BM), 1)
        out_b[...] = jnp.where(col > row, jax.nn.sigmoid(a), 0.0)


def _row_spec(w):
    return pl.BlockSpec((BR, w), lambda i: (i, 0))


def _full_spec(shape):
    nd = len(shape)
    return pl.BlockSpec(shape, lambda i: (0,) * nd)


def kernel(x, edge_index, mlp_W, mlp_b, W1, b1, W2, b2, Wm, bm, Ws, bs, noise):
    f32 = jnp.float32
    src = edge_index[0].astype(jnp.int32)
    dst = edge_index[1].astype(jnp.int32)
    mlp_b2 = mlp_b.reshape(1, H)
    b1_2 = b1.reshape(1, H)
    b2_2 = b2.reshape(1, H)
    bm_2 = bm.reshape(1, O)
    bs_2 = bs.reshape(1, O)
    zeros1 = jnp.zeros((N,), f32)
    zeros3 = jnp.zeros((NS, RPT, H), f32)

    deg_parts = _deg_kernel()(dst, zeros1)
    degp = deg_parts.T

    agg = _agg_kernel(H, CH)

    q1, dis = pl.pallas_call(
        _k1_body,
        grid=(GR,),
        in_specs=[_row_spec(D), _full_spec((D, H)), _full_spec((1, H)),
                  _full_spec((H, H)), pl.BlockSpec((BR, NC), lambda i: (i, 0))],
        out_specs=[_row_spec(H), pl.BlockSpec((BR, 1), lambda i: (i, 0))],
        out_shape=[jax.ShapeDtypeStruct((N, H), f32),
                   jax.ShapeDtypeStruct((N, 1), f32)],
    )(x, mlp_W, mlp_b2, W1, degp)

    s1 = agg(q1, src, dst, zeros3)

    def stage2(s, q, bias, W):
        return pl.pallas_call(
            _k2_body,
            grid=(GR,),
            in_specs=[pl.BlockSpec((NC, BR, H), lambda i: (0, i, 0)),
                      _row_spec(H), pl.BlockSpec((BR, 1), lambda i: (i, 0)),
                      _full_spec((1, H)), _full_spec((H, H))],
            out_specs=_row_spec(H),
            out_shape=jax.ShapeDtypeStruct((N, H), f32),
        )(s, q, dis, bias, W)

    q2 = stage2(s1, q1, b1_2, W2)
    s2 = agg(q2, src, dst, zeros3)

    u = pl.pallas_call(
        _k3_body,
        grid=(GR,),
        in_specs=[pl.BlockSpec((NC, BR, H), lambda i: (0, i, 0)),
                  _row_spec(H), pl.BlockSpec((BR, 1), lambda i: (i, 0)),
                  _full_spec((1, H))],
        out_specs=_row_spec(H),
        out_shape=jax.ShapeDtypeStruct((N, H), f32),
    )(s2, q2, dis, b2_2)

    s3 = agg(u, src, dst, zeros3)

    z = pl.pallas_call(
        _k4_body,
        grid=(GR,),
        in_specs=[pl.BlockSpec((NC, BR, H), lambda i: (0, i, 0)),
                  _row_spec(H), pl.BlockSpec((BR, 1), lambda i: (i, 0)),
                  _full_spec((H, O)), _full_spec((1, O)),
                  _full_spec((H, O)), _full_spec((1, O)),
                  _row_spec(O)],
        out_specs=_row_spec(O),
        out_shape=jax.ShapeDtypeStruct((N, O), f32),
    )(s3, u, dis, Wm, bm_2, Ws, bs_2, noise)

    zT = z.T

    adj = pl.pallas_call(
        _k5_body,
        grid=(GM, GM),
        in_specs=[pl.BlockSpec((BM, O), lambda i, j: (i, 0)),
                  pl.BlockSpec((O, BM), lambda i, j: (0, j))],
        out_specs=pl.BlockSpec((BM, BM), lambda i, j: (i, j)),
        out_shape=jax.ShapeDtypeStruct((N, N), f32),
    )(z, zT)

    return adj


# speedup vs baseline: 1.0899x; 1.0045x over previous
import functools

import jax
import jax.numpy as jnp
from jax import lax
from jax.experimental import pallas as pl
from jax.experimental.pallas import tpu as pltpu
from jax.experimental.pallas import tpu_sc as plsc

N = 10000
E = 320000
D = 128
H = 128
O = 16

NC = 2
NS = 16
EPC = E // NC
EPW = EPC // NS
CH = 40
FIRE = 5
NITER = EPW // (CH * FIRE)
NCH = EPW // CH
NPAD = 10240
RPT = NPAD // NS
RLAST = N - (NS - 1) * RPT


def _sc_mesh():
    return plsc.VectorSubcoreMesh(core_axis_name="c", subcore_axis_name="s")


@functools.lru_cache(maxsize=None)
def _deg_kernel():
    @functools.partial(
        pl.kernel,
        out_type=jax.ShapeDtypeStruct((NC, N), jnp.float32),
        mesh=_sc_mesh(),
        scratch_types=[
            [pltpu.VMEM((CH,), jnp.int32)] * FIRE,
            pltpu.VMEM((48,), jnp.float32),
            pltpu.VMEM_SHARED((N,), jnp.float32),
            [pltpu.SemaphoreType.DMA] * FIRE,
        ],
    )
    def deg_k(dst_hbm, zeros_hbm, out_hbm, didx_v, ones_v, acc_sh, dsems):
        cid = lax.axis_index("c")
        sid = lax.axis_index("s")
        wid = cid * NS + sid
        for k in range(3):
            ones_v[pl.ds(k * 16, 16)] = jnp.ones((16,), jnp.float32)
        @pl.when(sid == 0)
        def _zero():
            pltpu.sync_copy(zeros_hbm, acc_sh)

        plsc.subcore_barrier()
        ebase = wid * EPW

        def body(g, carry):
            base = ebase + g * (FIRE * CH)
            ddescs = [
                pltpu.async_copy(dst_hbm.at[pl.ds(base + b * CH, CH)],
                                 didx_v[b], dsems[b])
                for b in range(FIRE)
            ]
            for b in range(FIRE):
                ddescs[b].wait()
                pltpu.sync_copy(ones_v.at[pl.ds(0, CH)], acc_sh.at[didx_v[b]],
                                add=True)
            return carry

        lax.fori_loop(0, NCH // FIRE, body, 0)
        plsc.subcore_barrier()

        @pl.when(sid == 0)
        def _out():
            pltpu.sync_copy(acc_sh, out_hbm.at[cid])

    return deg_k


@functools.lru_cache(maxsize=None)
def _agg_kernel(w, ch):
    niter = EPW // (ch * FIRE)

    @functools.partial(
        pl.kernel,
        out_type=jax.ShapeDtypeStruct((NC, N, w), jnp.float32),
        mesh=_sc_mesh(),
        scratch_types=[
            pltpu.VMEM((EPW,), jnp.int32),
            pltpu.VMEM((EPW,), jnp.int32),
            [pltpu.VMEM((ch, w), jnp.float32)] * FIRE,
            pltpu.VMEM_SHARED((NPAD, w), jnp.float32),
            [pltpu.SemaphoreType.DMA] * FIRE,
            [pltpu.SemaphoreType.DMA] * FIRE,
            pltpu.SemaphoreType.DMA,
        ],
    )
    def agg_k(q_hbm, src_hbm, dst_hbm, zeros_hbm, out_hbm,
              sidx_v, didx_v, rows_v, acc_sh, sems, dsems, ssem):
        cid = lax.axis_index("c")
        sid = lax.axis_index("s")
        wid = cid * NS + sid
        r0 = pl.multiple_of(sid * RPT, 8)
        ebase = wid * EPW
        pltpu.sync_copy(zeros_hbm.at[sid], acc_sh.at[pl.ds(r0, RPT)])
        plsc.subcore_barrier()

        pltpu.sync_copy(src_hbm.at[pl.ds(ebase, EPW)], sidx_v)
        pltpu.sync_copy(dst_hbm.at[pl.ds(ebase, EPW)], didx_v)

        def body(g, carry):
            descs = [
                pltpu.async_copy(
                    q_hbm.at[sidx_v.at[pl.ds((g * FIRE + b) * ch, ch)]],
                    rows_v[b], sems[b])
                for b in range(FIRE)
            ]
            for b in range(FIRE):
                descs[b].wait()
                pltpu.sync_copy(
                    rows_v[b],
                    acc_sh.at[didx_v.at[pl.ds((g * FIRE + b) * ch, ch)]],
                    add=True)
            return carry

        lax.fori_loop(0, niter, body, 0)
        plsc.subcore_barrier()

        @pl.when(sid < NS - 1)
        def _out_full():
            pltpu.sync_copy(acc_sh.at[pl.ds(r0, RPT)],
                            out_hbm.at[cid, pl.ds(r0, RPT)])

        @pl.when(sid == NS - 1)
        def _out_last():
            pltpu.sync_copy(acc_sh.at[pl.ds((NS - 1) * RPT, RLAST)],
                            out_hbm.at[cid, pl.ds((NS - 1) * RPT, RLAST)])

    return agg_k


BR = 400
GR = N // BR

BM = 1024
GM = (N + BM - 1) // BM


def _k0_body(x_b, mlpW, mlpb, h0_b):
    h0_b[...] = jnp.dot(x_b[...], mlpW[...],
                        preferred_element_type=jnp.float32) + mlpb[...]


def _k1_body(h0_b, W1, degp, q1_b, dis_b):
    deg = degp[...][:, 0] + degp[...][:, 1] + 1.0
    dis = lax.rsqrt(deg)
    q1_b[...] = jnp.dot(h0_b[...], W1[...],
                        preferred_element_type=jnp.float32) * dis[:, None]
    dis_b[...] = dis[:, None]


def _k2_body(s_b, q_b, dis_b, bias, W, q2_b):
    agg = s_b[...][0] + s_b[...][1] + q_b[...]
    dis = dis_b[...]
    h = jnp.maximum(agg * dis + bias[...], 0.0)
    q2_b[...] = jnp.dot(h, W[...], preferred_element_type=jnp.float32) * dis


def _k3_body(s_b, q_b, dis_b, bias, u_b):
    agg = s_b[...][0] + s_b[...][1] + q_b[...]
    dis = dis_b[...]
    h = jnp.maximum(agg * dis + bias[...], 0.0)
    u_b[...] = h * dis


def _k4_body(s_b, u_b, dis_b, Wm, bm, Ws, bs, noise_b, z_b):
    g = (s_b[...][0] + s_b[...][1] + u_b[...]) * dis_b[...]
    mean = jnp.maximum(
        jnp.dot(g, Wm[...], preferred_element_type=jnp.float32) + bm[...], 0.0)
    logstd = jnp.maximum(
        jnp.dot(g, Ws[...], preferred_element_type=jnp.float32) + bs[...], 0.0)
    z_b[...] = noise_b[...] * jnp.exp(logstd) + mean


def _k5_body(z_b, zT_b, out_b):
    i = pl.program_id(0)
    j = pl.program_id(1)
    row0 = i * BM
    col0 = j * BM
    below = col0 + BM - 1 <= row0
    above = row0 + BM - 1 < col0

    @pl.when(below)
    def _zeros():
        out_b[...] = jnp.zeros((BM, BM), jnp.float32)

    @pl.when(above)
    def _full():
        a = jnp.dot(z_b[...], zT_b[...], preferred_element_type=jnp.float32)
        out_b[...] = jax.nn.sigmoid(a)

    @pl.when(jnp.logical_not(jnp.logical_or(below, above)))
    def _diag():
        a = jnp.dot(z_b[...], zT_b[...], preferred_element_type=jnp.float32)
        row = row0 + lax.broadcasted_iota(jnp.int32, (BM, BM), 0)
        col = col0 + lax.broadcasted_iota(jnp.int32, (BM, BM), 1)
        out_b[...] = jnp.where(col > row, jax.nn.sigmoid(a), 0.0)


def _row_spec(w):
    return pl.BlockSpec((BR, w), lambda i: (i, 0))


def _full_spec(shape):
    nd = len(shape)
    return pl.BlockSpec(shape, lambda i: (0,) * nd)


def kernel(x, edge_index, mlp_W, mlp_b, W1, b1, W2, b2, Wm, bm, Ws, bs, noise):
    f32 = jnp.float32
    src = edge_index[0].astype(jnp.int32)
    dst = edge_index[1].astype(jnp.int32)
    mlp_b2 = mlp_b.reshape(1, H)
    b1_2 = b1.reshape(1, H)
    b2_2 = b2.reshape(1, H)
    bm_2 = bm.reshape(1, O)
    bs_2 = bs.reshape(1, O)
    zeros1 = jnp.zeros((N,), f32)
    zeros3 = jnp.zeros((NS, RPT, H), f32)

    deg_parts = _deg_kernel()(dst, zeros1)
    degp = deg_parts.T

    agg = _agg_kernel(H, CH)

    h0 = pl.pallas_call(
        _k0_body,
        grid=(GR,),
        in_specs=[_row_spec(D), _full_spec((D, H)), _full_spec((1, H))],
        out_specs=_row_spec(H),
        out_shape=jax.ShapeDtypeStruct((N, H), f32),
    )(x, mlp_W, mlp_b2)

    q1, dis = pl.pallas_call(
        _k1_body,
        grid=(GR,),
        in_specs=[_row_spec(H), _full_spec((H, H)),
                  pl.BlockSpec((BR, NC), lambda i: (i, 0))],
        out_specs=[_row_spec(H), pl.BlockSpec((BR, 1), lambda i: (i, 0))],
        out_shape=[jax.ShapeDtypeStruct((N, H), f32),
                   jax.ShapeDtypeStruct((N, 1), f32)],
    )(h0, W1, degp)

    s1 = agg(q1, src, dst, zeros3)

    def stage2(s, q, bias, W):
        return pl.pallas_call(
            _k2_body,
            grid=(GR,),
            in_specs=[pl.BlockSpec((NC, BR, H), lambda i: (0, i, 0)),
                      _row_spec(H), pl.BlockSpec((BR, 1), lambda i: (i, 0)),
                      _full_spec((1, H)), _full_spec((H, H))],
            out_specs=_row_spec(H),
            out_shape=jax.ShapeDtypeStruct((N, H), f32),
        )(s, q, dis, bias, W)

    q2 = stage2(s1, q1, b1_2, W2)
    s2 = agg(q2, src, dst, zeros3)

    u = pl.pallas_call(
        _k3_body,
        grid=(GR,),
        in_specs=[pl.BlockSpec((NC, BR, H), lambda i: (0, i, 0)),
                  _row_spec(H), pl.BlockSpec((BR, 1), lambda i: (i, 0)),
                  _full_spec((1, H))],
        out_specs=_row_spec(H),
        out_shape=jax.ShapeDtypeStruct((N, H), f32),
    )(s2, q2, dis, b2_2)

    s3 = agg(u, src, dst, zeros3)

    z = pl.pallas_call(
        _k4_body,
        grid=(GR,),
        in_specs=[pl.BlockSpec((NC, BR, H), lambda i: (0, i, 0)),
                  _row_spec(H), pl.BlockSpec((BR, 1), lambda i: (i, 0)),
                  _full_spec((H, O)), _full_spec((1, O)),
                  _full_spec((H, O)), _full_spec((1, O)),
                  _row_spec(O)],
        out_specs=_row_spec(O),
        out_shape=jax.ShapeDtypeStruct((N, O), f32),
    )(s3, u, dis, Wm, bm_2, Ws, bs_2, noise)

    zT = z.T

    adj = pl.pallas_call(
        _k5_body,
        grid=(GM, GM),
        in_specs=[pl.BlockSpec((BM, O), lambda i, j: (i, 0)),
                  pl.BlockSpec((O, BM), lambda i, j: (0, j))],
        out_specs=pl.BlockSpec((BM, BM), lambda i, j: (i, j)),
        out_shape=jax.ShapeDtypeStruct((N, N), f32),
    )(z, zT)

    return adj


# speedup vs baseline: 1.1439x; 1.0495x over previous
import functools

import jax
import jax.numpy as jnp
from jax import lax
from jax.experimental import pallas as pl
from jax.experimental.pallas import tpu as pltpu
from jax.experimental.pallas import tpu_sc as plsc

N = 10000
E = 320000
D = 128
H = 128
O = 16

NC = 2
NS = 16
EPC = E // NC
EPW = EPC // NS
CH = 40
FIRE = 5
NITER = EPW // (CH * FIRE)
NCH = EPW // CH
NPAD = 10240
RPT = NPAD // NS
RLAST = N - (NS - 1) * RPT


def _sc_mesh():
    return plsc.VectorSubcoreMesh(core_axis_name="c", subcore_axis_name="s")


@functools.lru_cache(maxsize=None)
def _deg_kernel():
    @functools.partial(
        pl.kernel,
        out_type=jax.ShapeDtypeStruct((NC, N), jnp.float32),
        mesh=_sc_mesh(),
        scratch_types=[
            [pltpu.VMEM((CH,), jnp.int32)] * FIRE,
            pltpu.VMEM((48,), jnp.float32),
            pltpu.VMEM_SHARED((N,), jnp.float32),
            [pltpu.SemaphoreType.DMA] * FIRE,
        ],
    )
    def deg_k(dst_hbm, zeros_hbm, out_hbm, didx_v, ones_v, acc_sh, dsems):
        cid = lax.axis_index("c")
        sid = lax.axis_index("s")
        wid = cid * NS + sid
        for k in range(3):
            ones_v[pl.ds(k * 16, 16)] = jnp.ones((16,), jnp.float32)
        @pl.when(sid == 0)
        def _zero():
            pltpu.sync_copy(zeros_hbm, acc_sh)

        plsc.subcore_barrier()
        ebase = wid * EPW

        def body(g, carry):
            base = ebase + g * (FIRE * CH)
            ddescs = [
                pltpu.async_copy(dst_hbm.at[pl.ds(base + b * CH, CH)],
                                 didx_v[b], dsems[b])
                for b in range(FIRE)
            ]
            for b in range(FIRE):
                ddescs[b].wait()
                pltpu.sync_copy(ones_v.at[pl.ds(0, CH)], acc_sh.at[didx_v[b]],
                                add=True)
            return carry

        lax.fori_loop(0, NCH // FIRE, body, 0)
        plsc.subcore_barrier()

        @pl.when(sid == 0)
        def _out():
            pltpu.sync_copy(acc_sh, out_hbm.at[cid])

    return deg_k


@functools.lru_cache(maxsize=None)
def _agg_kernel(w, ch):
    niter = EPW // (ch * FIRE)

    @functools.partial(
        pl.kernel,
        out_type=jax.ShapeDtypeStruct((NC, N, w), jnp.float32),
        mesh=_sc_mesh(),
        scratch_types=[
            pltpu.VMEM((EPW,), jnp.int32),
            pltpu.VMEM((EPW,), jnp.int32),
            [pltpu.VMEM((ch, w), jnp.float32)] * FIRE,
            pltpu.VMEM_SHARED((NPAD, w), jnp.float32),
            [pltpu.SemaphoreType.DMA] * FIRE,
            [pltpu.SemaphoreType.DMA] * FIRE,
        ],
    )
    def agg_k(q_hbm, src_hbm, dst_hbm, zeros_hbm, out_hbm,
              sidx_v, didx_v, rows_v, acc_sh, sems, ssems):
        cid = lax.axis_index("c")
        sid = lax.axis_index("s")
        wid = cid * NS + sid
        r0 = pl.multiple_of(sid * RPT, 8)
        ebase = wid * EPW
        pltpu.sync_copy(zeros_hbm.at[sid], acc_sh.at[pl.ds(r0, RPT)])
        plsc.subcore_barrier()

        pltpu.sync_copy(src_hbm.at[pl.ds(ebase, EPW)], sidx_v)
        pltpu.sync_copy(dst_hbm.at[pl.ds(ebase, EPW)], didx_v)

        def _drain_scatters():
            for b in range(FIRE):
                pltpu.make_async_copy(q_hbm.at[pl.ds(0, ch)], rows_v[b],
                                      ssems[b]).wait()

        def body(g, carry):
            @pl.when(g > 0)
            def _reuse_guard():
                _drain_scatters()

            descs = [
                pltpu.async_copy(
                    q_hbm.at[sidx_v.at[pl.ds((g * FIRE + b) * ch, ch)]],
                    rows_v[b], sems[b])
                for b in range(FIRE)
            ]
            for b in range(FIRE):
                descs[b].wait()
                pltpu.async_copy(
                    rows_v[b],
                    acc_sh.at[didx_v.at[pl.ds((g * FIRE + b) * ch, ch)]],
                    ssems[b], add=True)
            return carry

        lax.fori_loop(0, niter, body, 0)
        _drain_scatters()
        plsc.subcore_barrier()

        @pl.when(sid < NS - 1)
        def _out_full():
            pltpu.sync_copy(acc_sh.at[pl.ds(r0, RPT)],
                            out_hbm.at[cid, pl.ds(r0, RPT)])

        @pl.when(sid == NS - 1)
        def _out_last():
            pltpu.sync_copy(acc_sh.at[pl.ds((NS - 1) * RPT, RLAST)],
                            out_hbm.at[cid, pl.ds((NS - 1) * RPT, RLAST)])

    return agg_k


BR = 400
GR = N // BR

BM = 1024
GM = (N + BM - 1) // BM


def _k0_body(x_b, mlpW, mlpb, h0_b):
    h0_b[...] = jnp.dot(x_b[...], mlpW[...],
                        preferred_element_type=jnp.float32) + mlpb[...]


def _k1_body(h0_b, W1, degp, q1_b, dis_b):
    deg = degp[...][:, 0] + degp[...][:, 1] + 1.0
    dis = lax.rsqrt(deg)
    q1_b[...] = jnp.dot(h0_b[...], W1[...],
                        preferred_element_type=jnp.float32) * dis[:, None]
    dis_b[...] = dis[:, None]


def _k2_body(s_b, q_b, dis_b, bias, W, q2_b):
    agg = s_b[...][0] + s_b[...][1] + q_b[...]
    dis = dis_b[...]
    h = jnp.maximum(agg * dis + bias[...], 0.0)
    q2_b[...] = jnp.dot(h, W[...], preferred_element_type=jnp.float32) * dis


def _k3_body(s_b, q_b, dis_b, bias, u_b):
    agg = s_b[...][0] + s_b[...][1] + q_b[...]
    dis = dis_b[...]
    h = jnp.maximum(agg * dis + bias[...], 0.0)
    u_b[...] = h * dis


def _k4_body(s_b, u_b, dis_b, Wm, bm, Ws, bs, noise_b, z_b):
    g = (s_b[...][0] + s_b[...][1] + u_b[...]) * dis_b[...]
    mean = jnp.maximum(
        jnp.dot(g, Wm[...], preferred_element_type=jnp.float32) + bm[...], 0.0)
    logstd = jnp.maximum(
        jnp.dot(g, Ws[...], preferred_element_type=jnp.float32) + bs[...], 0.0)
    z_b[...] = noise_b[...] * jnp.exp(logstd) + mean


def _k5_body(z_b, zT_b, out_b):
    i = pl.program_id(0)
    j = pl.program_id(1)
    row0 = i * BM
    col0 = j * BM
    below = col0 + BM - 1 <= row0
    above = row0 + BM - 1 < col0

    @pl.when(below)
    def _zeros():
        out_b[...] = jnp.zeros((BM, BM), jnp.float32)

    @pl.when(above)
    def _full():
        a = jnp.dot(z_b[...], zT_b[...], preferred_element_type=jnp.float32)
        out_b[...] = jax.nn.sigmoid(a)

    @pl.when(jnp.logical_not(jnp.logical_or(below, above)))
    def _diag():
        a = jnp.dot(z_b[...], zT_b[...], preferred_element_type=jnp.float32)
        row = row0 + lax.broadcasted_iota(jnp.int32, (BM, BM), 0)
        col = col0 + lax.broadcasted_iota(jnp.int32, (BM, BM), 1)
        out_b[...] = jnp.where(col > row, jax.nn.sigmoid(a), 0.0)


def _row_spec(w):
    return pl.BlockSpec((BR, w), lambda i: (i, 0))


def _full_spec(shape):
    nd = len(shape)
    return pl.BlockSpec(shape, lambda i: (0,) * nd)


def kernel(x, edge_index, mlp_W, mlp_b, W1, b1, W2, b2, Wm, bm, Ws, bs, noise):
    f32 = jnp.float32
    src = edge_index[0].astype(jnp.int32)
    dst = edge_index[1].astype(jnp.int32)
    mlp_b2 = mlp_b.reshape(1, H)
    b1_2 = b1.reshape(1, H)
    b2_2 = b2.reshape(1, H)
    bm_2 = bm.reshape(1, O)
    bs_2 = bs.reshape(1, O)
    zeros1 = jnp.zeros((N,), f32)
    zeros3 = jnp.zeros((NS, RPT, H), f32)

    deg_parts = _deg_kernel()(dst, zeros1)
    degp = deg_parts.T

    agg = _agg_kernel(H, CH)

    h0 = pl.pallas_call(
        _k0_body,
        grid=(GR,),
        in_specs=[_row_spec(D), _full_spec((D, H)), _full_spec((1, H))],
        out_specs=_row_spec(H),
        out_shape=jax.ShapeDtypeStruct((N, H), f32),
    )(x, mlp_W, mlp_b2)

    q1, dis = pl.pallas_call(
        _k1_body,
        grid=(GR,),
        in_specs=[_row_spec(H), _full_spec((H, H)),
                  pl.BlockSpec((BR, NC), lambda i: (i, 0))],
        out_specs=[_row_spec(H), pl.BlockSpec((BR, 1), lambda i: (i, 0))],
        out_shape=[jax.ShapeDtypeStruct((N, H), f32),
                   jax.ShapeDtypeStruct((N, 1), f32)],
    )(h0, W1, degp)

    s1 = agg(q1, src, dst, zeros3)

    def stage2(s, q, bias, W):
        return pl.pallas_call(
            _k2_body,
            grid=(GR,),
            in_specs=[pl.BlockSpec((NC, BR, H), lambda i: (0, i, 0)),
                      _row_spec(H), pl.BlockSpec((BR, 1), lambda i: (i, 0)),
                      _full_spec((1, H)), _full_spec((H, H))],
            out_specs=_row_spec(H),
            out_shape=jax.ShapeDtypeStruct((N, H), f32),
        )(s, q, dis, bias, W)

    q2 = stage2(s1, q1, b1_2, W2)
    s2 = agg(q2, src, dst, zeros3)

    u = pl.pallas_call(
        _k3_body,
        grid=(GR,),
        in_specs=[pl.BlockSpec((NC, BR, H), lambda i: (0, i, 0)),
                  _row_spec(H), pl.BlockSpec((BR, 1), lambda i: (i, 0)),
                  _full_spec((1, H))],
        out_specs=_row_spec(H),
        out_shape=jax.ShapeDtypeStruct((N, H), f32),
    )(s2, q2, dis, b2_2)

    s3 = agg(u, src, dst, zeros3)

    z = pl.pallas_call(
        _k4_body,
        grid=(GR,),
        in_specs=[pl.BlockSpec((NC, BR, H), lambda i: (0, i, 0)),
                  _row_spec(H), pl.BlockSpec((BR, 1), lambda i: (i, 0)),
                  _full_spec((H, O)), _full_spec((1, O)),
                  _full_spec((H, O)), _full_spec((1, O)),
                  _row_spec(O)],
        out_specs=_row_spec(O),
        out_shape=jax.ShapeDtypeStruct((N, O), f32),
    )(s3, u, dis, Wm, bm_2, Ws, bs_2, noise)

    zT = z.T

    adj = pl.pallas_call(
        _k5_body,
        grid=(GM, GM),
        in_specs=[pl.BlockSpec((BM, O), lambda i, j: (i, 0)),
                  pl.BlockSpec((O, BM), lambda i, j: (0, j))],
        out_specs=pl.BlockSpec((BM, BM), lambda i, j: (i, j)),
        out_shape=jax.ShapeDtypeStruct((N, N), f32),
    )(z, zT)

    return adj
